# initial kernel scaffold (unmeasured)
import jax
import jax.numpy as jnp
from jax import lax
from jax.experimental import pallas as pl
from jax.experimental.pallas import tpu as pltpu

N_DEV = 16
B_LOC = 2
SQ = 256
SKV = 256
HQ = 64
H_PER = 4
DH = 64
DM = 512


def kernel(x, Wq, K_ext, V_ext, Wo):
    my = lax.axis_index("i")
    b0 = my * B_LOC

    x2d = x.reshape(B_LOC * SQ, DM).astype(jnp.bfloat16)
    wq = Wq.astype(jnp.bfloat16)
    wo = Wo.astype(jnp.bfloat16)
    k_loc = lax.dynamic_slice_in_dim(K_ext, b0, B_LOC, axis=0)
    v_loc = lax.dynamic_slice_in_dim(V_ext, b0, B_LOC, axis=0)
    k_r = k_loc.transpose(0, 2, 1, 3).reshape(B_LOC * HQ, SKV, DH).astype(jnp.bfloat16)
    v_r = v_loc.transpose(0, 2, 1, 3).reshape(B_LOC * HQ, SKV, DH).astype(jnp.bfloat16)

    def body(x_ref, wq_ref, wo_ref, k_ref, v_ref, out_ref,
             commq, commo, ctx_ref, acc_ref, neg_ref,
             sendq, recvq, sendo, recvo):
        my_pos = lax.axis_index("i")
        left = lax.rem(my_pos + N_DEV - 1, N_DEV)
        right = lax.rem(my_pos + 1, N_DEV)

        barrier = pltpu.get_barrier_semaphore()
        for nbr in (left, right):
            pl.semaphore_signal(
                barrier, inc=1,
                device_id=(nbr,), device_id_type=pl.DeviceIdType.MESH,
            )
        pl.semaphore_wait(barrier, 2)

        qi = lax.broadcasted_iota(jnp.int32, (SQ, SKV), 0)
        ki = lax.broadcasted_iota(jnp.int32, (SQ, SKV), 1)
        mask = (jnp.abs(qi - ki) <= 128) | (ki < 32) | (qi < 32)
        neg_ref[...] = jnp.where(mask, 0.0, -1e9).astype(jnp.float32)

        def compute(slot):
            p = lax.rem(my_pos - slot + N_DEV, N_DEV)
            if slot == 0:
                wq_c = wq_ref[...]
                wo_c = wo_ref[...]
            else:
                wq_c = commq[slot, :, :]
                wo_c = commo[slot, :, :]
            q_all = lax.dot_general(
                x_ref[...], wq_c, (((1,), (0,)), ((), ())),
                preferred_element_type=jnp.float32,
            ).astype(jnp.bfloat16)
            for b in range(B_LOC):
                for hh in range(H_PER):
                    q = q_all[b * SQ:(b + 1) * SQ, hh * DH:(hh + 1) * DH]
                    idx = b * HQ + p * H_PER + hh
                    k = pl.load(k_ref, (pl.ds(idx, 1), slice(None), slice(None)))
                    v = pl.load(v_ref, (pl.ds(idx, 1), slice(None), slice(None)))
                    k = k.reshape(SKV, DH)
                    v = v.reshape(SKV, DH)
                    sc = lax.dot_general(
                        q, k, (((1,), (1,)), ((), ())),
                        preferred_element_type=jnp.float32,
                    ) * 0.125 + neg_ref[...]
                    m = jnp.max(sc, axis=1, keepdims=True)
                    e = jnp.exp(sc - m)
                    s = jnp.sum(e, axis=1, keepdims=True)
                    w = (e / s).astype(jnp.bfloat16)
                    ctx = lax.dot_general(
                        w, v, (((1,), (0,)), ((), ())),
                        preferred_element_type=jnp.float32,
                    )
                    ctx_ref[b * SQ:(b + 1) * SQ, hh * DH:(hh + 1) * DH] = ctx
            return lax.dot_general(
                ctx_ref[...].astype(jnp.bfloat16), wo_c,
                (((1,), (0,)), ((), ())),
                preferred_element_type=jnp.float32,
            )

        for h in range(1, N_DEV):
            src_q = wq_ref if h == 1 else commq.at[h - 1]
            src_o = wo_ref if h == 1 else commo.at[h - 1]
            rq = pltpu.make_async_remote_copy(
                src_ref=src_q, dst_ref=commq.at[h],
                send_sem=sendq.at[h], recv_sem=recvq.at[h],
                device_id=(right,), device_id_type=pl.DeviceIdType.MESH,
            )
            ro = pltpu.make_async_remote_copy(
                src_ref=src_o, dst_ref=commo.at[h],
                send_sem=sendo.at[h], recv_sem=recvo.at[h],
                device_id=(right,), device_id_type=pl.DeviceIdType.MESH,
            )
            rq.start()
            ro.start()
            if h == 1:
                acc_ref[...] = compute(0)
            else:
                acc_ref[...] = acc_ref[...] + compute(h - 1)
            rq.wait()
            ro.wait()
        acc_ref[...] = acc_ref[...] + compute(N_DEV - 1)

        out_ref[...] = acc_ref[...].reshape(B_LOC, SQ, DM)

    return pl.pallas_call(
        body,
        out_shape=jax.ShapeDtypeStruct((B_LOC, SQ, DM), jnp.float32),
        in_specs=[pl.BlockSpec(memory_space=pltpu.VMEM)] * 5,
        out_specs=pl.BlockSpec(memory_space=pltpu.VMEM),
        scratch_shapes=[
            pltpu.VMEM((N_DEV, DM, H_PER * DH), jnp.bfloat16),
            pltpu.VMEM((N_DEV, H_PER * DH, DM), jnp.bfloat16),
            pltpu.VMEM((B_LOC * SQ, H_PER * DH), jnp.float32),
            pltpu.VMEM((B_LOC * SQ, DM), jnp.float32),
            pltpu.VMEM((SQ, SKV), jnp.float32),
            pltpu.SemaphoreType.DMA((N_DEV,)),
            pltpu.SemaphoreType.DMA((N_DEV,)),
            pltpu.SemaphoreType.DMA((N_DEV,)),
            pltpu.SemaphoreType.DMA((N_DEV,)),
        ],
        compiler_params=pltpu.CompilerParams(collective_id=0),
    )(x2d, wq, wo, k_r, v_r)


# baseline (device time: 133783 ns/iter reference)
import jax
import jax.numpy as jnp
from jax import lax
from jax.experimental import pallas as pl
from jax.experimental.pallas import tpu as pltpu

N_DEV = 16
B_LOC = 2
SQ = 256
SKV = 256
HQ = 64
H_PER = 4
DH = 64
DM = 512


def kernel(x, Wq, K_ext, V_ext, Wo):
    my = lax.axis_index("i")
    b0 = my * B_LOC

    x2d = x.reshape(B_LOC * SQ, DM).astype(jnp.bfloat16)
    wq = Wq.astype(jnp.bfloat16)
    wo = Wo.astype(jnp.bfloat16)
    k_loc = lax.dynamic_slice_in_dim(K_ext, b0, B_LOC, axis=0)
    v_loc = lax.dynamic_slice_in_dim(V_ext, b0, B_LOC, axis=0)
    k_r = k_loc.transpose(0, 2, 1, 3).reshape(B_LOC * HQ, SKV, DH).astype(jnp.bfloat16)
    v_r = v_loc.transpose(0, 2, 1, 3).reshape(B_LOC * HQ, SKV, DH).astype(jnp.bfloat16)

    def body(x_ref, wq_ref, wo_ref, k_ref, v_ref, out_ref,
             commq, commo, ctx_ref, acc_ref, neg_ref,
             sendq, recvq, sendo, recvo):
        my_pos = lax.axis_index("i")
        left = lax.rem(my_pos + N_DEV - 1, N_DEV)
        right = lax.rem(my_pos + 1, N_DEV)

        barrier = pltpu.get_barrier_semaphore()
        for nbr in (left, right):
            pl.semaphore_signal(
                barrier, inc=1,
                device_id=(nbr,), device_id_type=pl.DeviceIdType.MESH,
            )
        pl.semaphore_wait(barrier, 2)

        qi = lax.broadcasted_iota(jnp.int32, (SQ, SKV), 0)
        ki = lax.broadcasted_iota(jnp.int32, (SQ, SKV), 1)
        mask = (jnp.abs(qi - ki) <= 128) | (ki < 32) | (qi < 32)
        neg_ref[...] = jnp.where(mask, 0.0, -1e9).astype(jnp.float32)

        def compute(slot):
            p = lax.rem(my_pos - slot + N_DEV, N_DEV)
            if slot == 0:
                wq_c = wq_ref[...]
                wo_c = wo_ref[...]
            else:
                wq_c = commq[slot, :, :]
                wo_c = commo[slot, :, :]
            q_all = lax.dot_general(
                x_ref[...], wq_c, (((1,), (0,)), ((), ())),
                preferred_element_type=jnp.float32,
            ).astype(jnp.bfloat16)
            for b in range(B_LOC):
                for hh in range(H_PER):
                    q = q_all[b * SQ:(b + 1) * SQ, hh * DH:(hh + 1) * DH]
                    idx = b * HQ + p * H_PER + hh
                    k = k_ref[pl.ds(idx, 1), :, :].reshape(SKV, DH)
                    v = v_ref[pl.ds(idx, 1), :, :].reshape(SKV, DH)
                    sc = lax.dot_general(
                        q, k, (((1,), (1,)), ((), ())),
                        preferred_element_type=jnp.float32,
                    ) * 0.125 + neg_ref[...]
                    m = jnp.max(sc, axis=1, keepdims=True)
                    e = jnp.exp(sc - m)
                    s = jnp.sum(e, axis=1, keepdims=True)
                    w = (e / s).astype(jnp.bfloat16)
                    ctx = lax.dot_general(
                        w, v, (((1,), (0,)), ((), ())),
                        preferred_element_type=jnp.float32,
                    )
                    ctx_ref[b * SQ:(b + 1) * SQ, hh * DH:(hh + 1) * DH] = ctx
            return lax.dot_general(
                ctx_ref[...].astype(jnp.bfloat16), wo_c,
                (((1,), (0,)), ((), ())),
                preferred_element_type=jnp.float32,
            )

        for h in range(1, N_DEV):
            src_q = wq_ref if h == 1 else commq.at[h - 1]
            src_o = wo_ref if h == 1 else commo.at[h - 1]
            rq = pltpu.make_async_remote_copy(
                src_ref=src_q, dst_ref=commq.at[h],
                send_sem=sendq.at[h], recv_sem=recvq.at[h],
                device_id=(right,), device_id_type=pl.DeviceIdType.MESH,
            )
            ro = pltpu.make_async_remote_copy(
                src_ref=src_o, dst_ref=commo.at[h],
                send_sem=sendo.at[h], recv_sem=recvo.at[h],
                device_id=(right,), device_id_type=pl.DeviceIdType.MESH,
            )
            rq.start()
            ro.start()
            if h == 1:
                acc_ref[...] = compute(0)
            else:
                acc_ref[...] = acc_ref[...] + compute(h - 1)
            rq.wait()
            ro.wait()
        acc_ref[...] = acc_ref[...] + compute(N_DEV - 1)

        out_ref[...] = acc_ref[...].reshape(B_LOC, SQ, DM)

    return pl.pallas_call(
        body,
        out_shape=jax.ShapeDtypeStruct((B_LOC, SQ, DM), jnp.float32),
        in_specs=[pl.BlockSpec(memory_space=pltpu.VMEM)] * 5,
        out_specs=pl.BlockSpec(memory_space=pltpu.VMEM),
        scratch_shapes=[
            pltpu.VMEM((N_DEV, DM, H_PER * DH), jnp.bfloat16),
            pltpu.VMEM((N_DEV, H_PER * DH, DM), jnp.bfloat16),
            pltpu.VMEM((B_LOC * SQ, H_PER * DH), jnp.float32),
            pltpu.VMEM((B_LOC * SQ, DM), jnp.float32),
            pltpu.VMEM((SQ, SKV), jnp.float32),
            pltpu.SemaphoreType.DMA((N_DEV,)),
            pltpu.SemaphoreType.DMA((N_DEV,)),
            pltpu.SemaphoreType.DMA((N_DEV,)),
            pltpu.SemaphoreType.DMA((N_DEV,)),
        ],
        compiler_params=pltpu.CompilerParams(collective_id=0),
    )(x2d, wq, wo, k_r, v_r)


# device time: 94023 ns/iter; 1.4229x vs baseline; 1.4229x over previous
import jax
import jax.numpy as jnp
from jax import lax
from jax.experimental import pallas as pl
from jax.experimental.pallas import tpu as pltpu

N_DEV = 16
NF = 8
NB = 7
B_LOC = 2
SQ = 256
SKV = 256
HQ = 64
H_PER = 4
DH = 64
DM = 512


def kernel(x, Wq, K_ext, V_ext, Wo):
    my = lax.axis_index("i")
    b0 = my * B_LOC

    x2d = x.reshape(B_LOC * SQ, DM).astype(jnp.bfloat16)
    wq = Wq.astype(jnp.bfloat16)
    wo = Wo.astype(jnp.bfloat16)
    k_loc = lax.dynamic_slice_in_dim(K_ext, b0, B_LOC, axis=0)
    v_loc = lax.dynamic_slice_in_dim(V_ext, b0, B_LOC, axis=0)
    k_r = k_loc.transpose(0, 2, 1, 3).reshape(B_LOC * HQ, SKV, DH).astype(jnp.bfloat16)
    v_r = v_loc.transpose(0, 2, 1, 3).reshape(B_LOC * HQ, SKV, DH).astype(jnp.bfloat16)

    def body(x_ref, wq_ref, wo_ref, k_ref, v_ref, out_ref,
             commq_f, commo_f, commq_b, commo_b, ctx_ref, acc_ref, neg_ref,
             sendq_f, recvq_f, sendo_f, recvo_f,
             sendq_b, recvq_b, sendo_b, recvo_b):
        my_pos = lax.axis_index("i")
        left = lax.rem(my_pos + N_DEV - 1, N_DEV)
        right = lax.rem(my_pos + 1, N_DEV)

        barrier = pltpu.get_barrier_semaphore()
        for nbr in (left, right):
            pl.semaphore_signal(
                barrier, inc=1,
                device_id=(nbr,), device_id_type=pl.DeviceIdType.MESH,
            )
        pl.semaphore_wait(barrier, 2)

        qi = lax.broadcasted_iota(jnp.int32, (SQ, SKV), 0)
        ki = lax.broadcasted_iota(jnp.int32, (SQ, SKV), 1)
        mask = (jnp.abs(qi - ki) <= 128) | (ki < 32) | (qi < 32)
        neg_ref[...] = jnp.where(mask, 0.0, -1e9).astype(jnp.float32)

        def compute(p, wq_c, wo_c):
            q_all = lax.dot_general(
                x_ref[...], wq_c, (((1,), (0,)), ((), ())),
                preferred_element_type=jnp.float32,
            ).astype(jnp.bfloat16)
            for b in range(B_LOC):
                for hh in range(H_PER):
                    q = q_all[b * SQ:(b + 1) * SQ, hh * DH:(hh + 1) * DH]
                    idx = b * HQ + p * H_PER + hh
                    k = k_ref[pl.ds(idx, 1), :, :].reshape(SKV, DH)
                    v = v_ref[pl.ds(idx, 1), :, :].reshape(SKV, DH)
                    sc = lax.dot_general(
                        q, k, (((1,), (1,)), ((), ())),
                        preferred_element_type=jnp.float32,
                    ) * 0.125 + neg_ref[...]
                    m = jnp.max(sc, axis=1, keepdims=True)
                    e = jnp.exp(sc - m)
                    s = jnp.sum(e, axis=1, keepdims=True)
                    w = (e / s).astype(jnp.bfloat16)
                    ctx = lax.dot_general(
                        w, v, (((1,), (0,)), ((), ())),
                        preferred_element_type=jnp.float32,
                    )
                    ctx_ref[b * SQ:(b + 1) * SQ, hh * DH:(hh + 1) * DH] = ctx
            return lax.dot_general(
                ctx_ref[...].astype(jnp.bfloat16), wo_c,
                (((1,), (0,)), ((), ())),
                preferred_element_type=jnp.float32,
            )

        for s in range(1, NF + 1):
            sqf = wq_ref if s == 1 else commq_f.at[s - 1]
            sof = wo_ref if s == 1 else commo_f.at[s - 1]
            rqf = pltpu.make_async_remote_copy(
                src_ref=sqf, dst_ref=commq_f.at[s],
                send_sem=sendq_f.at[s], recv_sem=recvq_f.at[s],
                device_id=(right,), device_id_type=pl.DeviceIdType.MESH,
            )
            rof = pltpu.make_async_remote_copy(
                src_ref=sof, dst_ref=commo_f.at[s],
                send_sem=sendo_f.at[s], recv_sem=recvo_f.at[s],
                device_id=(right,), device_id_type=pl.DeviceIdType.MESH,
            )
            rqf.start()
            rof.start()
            if s <= NB:
                sqb = wq_ref if s == 1 else commq_b.at[s - 1]
                sob = wo_ref if s == 1 else commo_b.at[s - 1]
                rqb = pltpu.make_async_remote_copy(
                    src_ref=sqb, dst_ref=commq_b.at[s],
                    send_sem=sendq_b.at[s], recv_sem=recvq_b.at[s],
                    device_id=(left,), device_id_type=pl.DeviceIdType.MESH,
                )
                rob = pltpu.make_async_remote_copy(
                    src_ref=sob, dst_ref=commo_b.at[s],
                    send_sem=sendo_b.at[s], recv_sem=recvo_b.at[s],
                    device_id=(left,), device_id_type=pl.DeviceIdType.MESH,
                )
                rqb.start()
                rob.start()
            if s == 1:
                acc_ref[...] = compute(my_pos, wq_ref[...], wo_ref[...])
            else:
                pf = lax.rem(my_pos - (s - 1) + N_DEV, N_DEV)
                af = compute(pf, commq_f[s - 1, :, :], commo_f[s - 1, :, :])
                pb = lax.rem(my_pos + (s - 1), N_DEV)
                ab = compute(pb, commq_b[s - 1, :, :], commo_b[s - 1, :, :])
                acc_ref[...] = acc_ref[...] + af + ab
            rqf.wait()
            rof.wait()
            if s <= NB:
                rqb.wait()
                rob.wait()

        pf = lax.rem(my_pos - NF + N_DEV, N_DEV)
        acc_ref[...] = acc_ref[...] + compute(pf, commq_f[NF, :, :], commo_f[NF, :, :])

        out_ref[...] = acc_ref[...].reshape(B_LOC, SQ, DM)

    return pl.pallas_call(
        body,
        out_shape=jax.ShapeDtypeStruct((B_LOC, SQ, DM), jnp.float32),
        in_specs=[pl.BlockSpec(memory_space=pltpu.VMEM)] * 5,
        out_specs=pl.BlockSpec(memory_space=pltpu.VMEM),
        scratch_shapes=[
            pltpu.VMEM((NF + 1, DM, H_PER * DH), jnp.bfloat16),
            pltpu.VMEM((NF + 1, H_PER * DH, DM), jnp.bfloat16),
            pltpu.VMEM((NB + 1, DM, H_PER * DH), jnp.bfloat16),
            pltpu.VMEM((NB + 1, H_PER * DH, DM), jnp.bfloat16),
            pltpu.VMEM((B_LOC * SQ, H_PER * DH), jnp.float32),
            pltpu.VMEM((B_LOC * SQ, DM), jnp.float32),
            pltpu.VMEM((SQ, SKV), jnp.float32),
            pltpu.SemaphoreType.DMA((NF + 1,)),
            pltpu.SemaphoreType.DMA((NF + 1,)),
            pltpu.SemaphoreType.DMA((NF + 1,)),
            pltpu.SemaphoreType.DMA((NF + 1,)),
            pltpu.SemaphoreType.DMA((NB + 1,)),
            pltpu.SemaphoreType.DMA((NB + 1,)),
            pltpu.SemaphoreType.DMA((NB + 1,)),
            pltpu.SemaphoreType.DMA((NB + 1,)),
        ],
        compiler_params=pltpu.CompilerParams(collective_id=0),
    )(x2d, wq, wo, k_r, v_r)


# device time: 93658 ns/iter; 1.4284x vs baseline; 1.0039x over previous
import jax
import jax.numpy as jnp
from jax import lax
from jax.experimental import pallas as pl
from jax.experimental.pallas import tpu as pltpu

N_DEV = 16
NF = 8
NB = 7
B_LOC = 2
SQ = 256
SKV = 256
HQ = 64
H_PER = 4
DH = 64
DM = 512


def kernel(x, Wq, K_ext, V_ext, Wo):
    my = lax.axis_index("i")
    b0 = my * B_LOC

    x2d = (x.reshape(B_LOC * SQ, DM) * 0.125).astype(jnp.bfloat16)
    wq = Wq.astype(jnp.bfloat16)
    wo = Wo.astype(jnp.bfloat16)
    k_loc = lax.dynamic_slice_in_dim(K_ext, b0, B_LOC, axis=0)
    v_loc = lax.dynamic_slice_in_dim(V_ext, b0, B_LOC, axis=0)
    k_r = k_loc.transpose(0, 2, 1, 3).reshape(B_LOC * HQ, SKV, DH).astype(jnp.bfloat16)
    v_r = v_loc.transpose(0, 2, 1, 3).reshape(B_LOC * HQ, SKV, DH).astype(jnp.bfloat16)

    def body(x_ref, wq_ref, wo_ref, k_ref, v_ref, out_ref,
             commq_f, commo_f, commq_b, commo_b, ctx_ref, acc_ref, neg_ref,
             sendq_f, recvq_f, sendo_f, recvo_f,
             sendq_b, recvq_b, sendo_b, recvo_b):
        my_pos = lax.axis_index("i")
        left = lax.rem(my_pos + N_DEV - 1, N_DEV)
        right = lax.rem(my_pos + 1, N_DEV)

        barrier = pltpu.get_barrier_semaphore()
        for nbr in (left, right):
            pl.semaphore_signal(
                barrier, inc=1,
                device_id=(nbr,), device_id_type=pl.DeviceIdType.MESH,
            )
        pl.semaphore_wait(barrier, 2)

        qi = lax.broadcasted_iota(jnp.int32, (SQ, SKV), 0)
        ki = lax.broadcasted_iota(jnp.int32, (SQ, SKV), 1)
        mask = (jnp.abs(qi - ki) <= 128) | (ki < 32) | (qi < 32)
        neg_ref[...] = jnp.where(mask, 0.0, -1e9).astype(jnp.float32)

        def compute(p, wq_c, wo_c):
            q_all = lax.dot_general(
                x_ref[...], wq_c, (((1,), (0,)), ((), ())),
                preferred_element_type=jnp.float32,
            ).astype(jnp.bfloat16)
            for b in range(B_LOC):
                for hh in range(H_PER):
                    q = q_all[b * SQ:(b + 1) * SQ, hh * DH:(hh + 1) * DH]
                    idx = b * HQ + p * H_PER + hh
                    k = k_ref[pl.ds(idx, 1), :, :].reshape(SKV, DH)
                    v = v_ref[pl.ds(idx, 1), :, :].reshape(SKV, DH)
                    sc = lax.dot_general(
                        q, k, (((1,), (1,)), ((), ())),
                        preferred_element_type=jnp.float32,
                    ) + neg_ref[...]
                    e = jnp.exp(sc)
                    s = jnp.sum(e, axis=1, keepdims=True)
                    ctx = lax.dot_general(
                        e.astype(jnp.bfloat16), v, (((1,), (0,)), ((), ())),
                        preferred_element_type=jnp.float32,
                    ) * (1.0 / s)
                    ctx_ref[b * SQ:(b + 1) * SQ, hh * DH:(hh + 1) * DH] = ctx
            return lax.dot_general(
                ctx_ref[...].astype(jnp.bfloat16), wo_c,
                (((1,), (0,)), ((), ())),
                preferred_element_type=jnp.float32,
            )

        for s in range(1, NF + 1):
            sqf = wq_ref if s == 1 else commq_f.at[s - 1]
            sof = wo_ref if s == 1 else commo_f.at[s - 1]
            rqf = pltpu.make_async_remote_copy(
                src_ref=sqf, dst_ref=commq_f.at[s],
                send_sem=sendq_f.at[s], recv_sem=recvq_f.at[s],
                device_id=(right,), device_id_type=pl.DeviceIdType.MESH,
            )
            rof = pltpu.make_async_remote_copy(
                src_ref=sof, dst_ref=commo_f.at[s],
                send_sem=sendo_f.at[s], recv_sem=recvo_f.at[s],
                device_id=(right,), device_id_type=pl.DeviceIdType.MESH,
            )
            rqf.start()
            rof.start()
            if s <= NB:
                sqb = wq_ref if s == 1 else commq_b.at[s - 1]
                sob = wo_ref if s == 1 else commo_b.at[s - 1]
                rqb = pltpu.make_async_remote_copy(
                    src_ref=sqb, dst_ref=commq_b.at[s],
                    send_sem=sendq_b.at[s], recv_sem=recvq_b.at[s],
                    device_id=(left,), device_id_type=pl.DeviceIdType.MESH,
                )
                rob = pltpu.make_async_remote_copy(
                    src_ref=sob, dst_ref=commo_b.at[s],
                    send_sem=sendo_b.at[s], recv_sem=recvo_b.at[s],
                    device_id=(left,), device_id_type=pl.DeviceIdType.MESH,
                )
                rqb.start()
                rob.start()
            if s == 1:
                acc_ref[...] = compute(my_pos, wq_ref[...], wo_ref[...])
            else:
                pf = lax.rem(my_pos - (s - 1) + N_DEV, N_DEV)
                af = compute(pf, commq_f[s - 1, :, :], commo_f[s - 1, :, :])
                pb = lax.rem(my_pos + (s - 1), N_DEV)
                ab = compute(pb, commq_b[s - 1, :, :], commo_b[s - 1, :, :])
                acc_ref[...] = acc_ref[...] + af + ab
            rqf.wait()
            rof.wait()
            if s <= NB:
                rqb.wait()
                rob.wait()

        pf = lax.rem(my_pos - NF + N_DEV, N_DEV)
        acc_ref[...] = acc_ref[...] + compute(pf, commq_f[NF, :, :], commo_f[NF, :, :])

        out_ref[...] = acc_ref[...].reshape(B_LOC, SQ, DM)

    return pl.pallas_call(
        body,
        out_shape=jax.ShapeDtypeStruct((B_LOC, SQ, DM), jnp.float32),
        in_specs=[pl.BlockSpec(memory_space=pltpu.VMEM)] * 5,
        out_specs=pl.BlockSpec(memory_space=pltpu.VMEM),
        scratch_shapes=[
            pltpu.VMEM((NF + 1, DM, H_PER * DH), jnp.bfloat16),
            pltpu.VMEM((NF + 1, H_PER * DH, DM), jnp.bfloat16),
            pltpu.VMEM((NB + 1, DM, H_PER * DH), jnp.bfloat16),
            pltpu.VMEM((NB + 1, H_PER * DH, DM), jnp.bfloat16),
            pltpu.VMEM((B_LOC * SQ, H_PER * DH), jnp.float32),
            pltpu.VMEM((B_LOC * SQ, DM), jnp.float32),
            pltpu.VMEM((SQ, SKV), jnp.float32),
            pltpu.SemaphoreType.DMA((NF + 1,)),
            pltpu.SemaphoreType.DMA((NF + 1,)),
            pltpu.SemaphoreType.DMA((NF + 1,)),
            pltpu.SemaphoreType.DMA((NF + 1,)),
            pltpu.SemaphoreType.DMA((NB + 1,)),
            pltpu.SemaphoreType.DMA((NB + 1,)),
            pltpu.SemaphoreType.DMA((NB + 1,)),
            pltpu.SemaphoreType.DMA((NB + 1,)),
        ],
        compiler_params=pltpu.CompilerParams(collective_id=0),
    )(x2d, wq, wo, k_r, v_r)
